# baseline (speedup 1.0000x reference)
import functools

import jax
import jax.numpy as jnp
import numpy as np
from jax import lax
from jax.experimental import pallas as pl
from jax.experimental.pallas import tpu as pltpu
from jax.experimental.pallas import tpu_sc as plsc

N_ELEMENTS = 1000000
DIM = 32
MAX_LEN = 200
B = 4096
L = 200

NC = 2
NS = 16
NW = NC * NS

TOTAL = B * L
ROWS_PER_W = TOTAL // NW
SEQS_PER_CHUNK = 8
CHUNK = SEQS_PER_CHUNK * L
NCHUNKS = ROWS_PER_W // CHUNK
VPER = CHUNK // 16

K1 = 1024
NBLK1 = 977
NPAD = NBLK1 * K1


def _sinusoidal_pe():
    pos = np.arange(MAX_LEN, dtype=np.float32)[:, None]
    div = np.exp(np.arange(0, DIM, 2, dtype=np.float32) * (-np.log(10000.0) / DIM))
    pe = np.zeros((MAX_LEN, DIM), dtype=np.float32)
    pe[:, 0::2] = np.sin(pos * div)
    pe[:, 1::2] = np.cos(pos * div)
    return pe


_PE = _sinusoidal_pe()


def _tc_body(x_ref, y_ref):
    x = x_ref[...]
    parts = [x[:, k * (K1 // 4):(k + 1) * (K1 // 4)].T for k in range(4)]
    y_ref[...] = jnp.concatenate(parts, axis=1)


def _relayout_table(tT):
    return pl.pallas_call(
        _tc_body,
        grid=(NBLK1,),
        in_specs=[pl.BlockSpec((32, K1), lambda i: (0, i))],
        out_specs=pl.BlockSpec((K1 // 4, 128), lambda i: (i, 0)),
        out_shape=jax.ShapeDtypeStruct((NPAD // 4, 128), jnp.float32),
    )(tT)


def _sc_body(table_hbm, idx_hbm, pe_hbm, out_hbm, idx_v, m_v, pe_v, rows_v, sem):
    wid = lax.axis_index("s") * NC + lax.axis_index("c")
    base = wid * ROWS_PER_W

    pltpu.sync_copy(pe_hbm, pe_v)

    def chunk_body(g, carry):
        start = base + g * CHUNK
        pltpu.sync_copy(idx_hbm.at[pl.ds(start, CHUNK)], idx_v)

        def m_body(q, c):
            r = idx_v[pl.ds(q * 16, 16)]
            m = ((r & -1024)
                 | lax.shift_left(r & 255, 2)
                 | (lax.shift_right_logical(r, 8) & 3))
            m_v[pl.ds(q * 16, 16)] = m
            return c

        lax.fori_loop(0, VPER, m_body, 0, unroll=False)

        pltpu.async_copy(table_hbm.at[m_v], rows_v, sem).wait()

        def pe_body(j, c):
            p0 = pe_v[j, pl.ds(0, 16)]
            p1 = pe_v[j, pl.ds(16, 16)]
            for s in range(SEQS_PER_CHUNK):
                r = s * L + j
                rows_v[r, pl.ds(0, 16)] = rows_v[r, pl.ds(0, 16)] + p0
                rows_v[r, pl.ds(16, 16)] = rows_v[r, pl.ds(16, 16)] + p1
            return c

        lax.fori_loop(0, L, pe_body, 0, unroll=False)

        pltpu.sync_copy(rows_v, out_hbm.at[pl.ds(start, CHUNK)])
        return carry

    lax.fori_loop(0, NCHUNKS, chunk_body, 0, unroll=False)


@jax.jit
def _run(kb_ids_seq, key_emb_table):
    table_lin = _relayout_table(key_emb_table.T).reshape(NPAD, DIM)
    idx_flat = kb_ids_seq.reshape(TOTAL)
    mesh = plsc.VectorSubcoreMesh(core_axis_name="c", subcore_axis_name="s")
    f = pl.kernel(
        _sc_body,
        out_type=jax.ShapeDtypeStruct((TOTAL, DIM), jnp.float32),
        mesh=mesh,
        scratch_types=[
            pltpu.VMEM((CHUNK,), jnp.int32),
            pltpu.VMEM((CHUNK,), jnp.int32),
            pltpu.VMEM((MAX_LEN, DIM), jnp.float32),
            pltpu.VMEM((CHUNK, DIM), jnp.float32),
            pltpu.SemaphoreType.DMA,
        ],
        compiler_params=pltpu.CompilerParams(use_tc_tiling_on_sc=False),
    )
    out = f(table_lin, idx_flat, jnp.asarray(_PE))
    return out.reshape(B, L, DIM)


def kernel(kb_ids_seq, key_emb_table):
    return _run(kb_ids_seq, key_emb_table)


# speedup vs baseline: 1.4801x; 1.4801x over previous
import functools

import jax
import jax.numpy as jnp
import numpy as np
from jax import lax
from jax.experimental import pallas as pl
from jax.experimental.pallas import tpu as pltpu
from jax.experimental.pallas import tpu_sc as plsc

N_ELEMENTS = 1000000
DIM = 32
MAX_LEN = 200
B = 4096
L = 200

NC = 2
NS = 16
NW = NC * NS

TOTAL = B * L
ROWS_PER_W = TOTAL // NW
SEQS_PER_CHUNK = 8
CHUNK = SEQS_PER_CHUNK * L
NCHUNKS = ROWS_PER_W // CHUNK
VPER = CHUNK // 16

K1 = 4096
Q1 = K1 // 4
NBLK1 = 245
NPAD = NBLK1 * K1


def _sinusoidal_pe():
    pos = np.arange(MAX_LEN, dtype=np.float32)[:, None]
    div = np.exp(np.arange(0, DIM, 2, dtype=np.float32) * (-np.log(10000.0) / DIM))
    pe = np.zeros((MAX_LEN, DIM), dtype=np.float32)
    pe[:, 0::2] = np.sin(pos * div)
    pe[:, 1::2] = np.cos(pos * div)
    return pe


_PE = _sinusoidal_pe()


def _selectors():
    e = np.zeros((4, 32, 128), dtype=np.float32)
    for k in range(4):
        for c in range(32):
            e[k, c, 32 * k + c] = 1.0
    return e


_E = _selectors()


def _tc_body(x_ref, e_ref, y_ref):
    x = x_ref[...]
    gcol = pl.program_id(0) * K1 + lax.broadcasted_iota(jnp.int32, (32, K1), 1)
    x = jnp.where(gcol < N_ELEMENTS, x, 0.0)
    acc = jnp.zeros((Q1, 128), jnp.float32)
    for k in range(4):
        xk = x[:, k * Q1:(k + 1) * Q1]
        acc = acc + lax.dot_general(
            xk, e_ref[k], (((0,), (0,)), ((), ())),
            preferred_element_type=jnp.float32)
    y_ref[...] = acc


def _relayout_table(tT, e):
    return pl.pallas_call(
        _tc_body,
        grid=(NBLK1,),
        in_specs=[
            pl.BlockSpec((32, K1), lambda i: (0, i)),
            pl.BlockSpec((4, 32, 128), lambda i: (0, 0, 0)),
        ],
        out_specs=pl.BlockSpec((Q1, 128), lambda i: (i, 0)),
        out_shape=jax.ShapeDtypeStruct((NPAD // 4, 128), jnp.float32),
    )(tT, e)


def _sc_body(table_hbm, idx_hbm, pe_hbm, out_hbm, idx_v, m_v, pe_v, rows_v, sem):
    wid = lax.axis_index("s") * NC + lax.axis_index("c")
    base = wid * ROWS_PER_W

    pltpu.sync_copy(pe_hbm, pe_v)

    def chunk_body(g, carry):
        start = base + g * CHUNK
        pltpu.sync_copy(idx_hbm.at[pl.ds(start, CHUNK)], idx_v)

        def m_body(q, c):
            r = idx_v[pl.ds(q * 16, 16)]
            m = ((r & -K1)
                 | lax.shift_left(r & (Q1 - 1), 2)
                 | (lax.shift_right_logical(r, 10) & 3))
            m_v[pl.ds(q * 16, 16)] = m
            return c

        lax.fori_loop(0, VPER, m_body, 0, unroll=False)

        pltpu.async_copy(table_hbm.at[m_v], rows_v, sem).wait()

        def pe_body(j, c):
            p0 = pe_v[j, pl.ds(0, 16)]
            p1 = pe_v[j, pl.ds(16, 16)]
            for s in range(SEQS_PER_CHUNK):
                r = s * L + j
                rows_v[r, pl.ds(0, 16)] = rows_v[r, pl.ds(0, 16)] + p0
                rows_v[r, pl.ds(16, 16)] = rows_v[r, pl.ds(16, 16)] + p1
            return c

        lax.fori_loop(0, L, pe_body, 0, unroll=False)

        pltpu.sync_copy(rows_v, out_hbm.at[pl.ds(start, CHUNK)])
        return carry

    lax.fori_loop(0, NCHUNKS, chunk_body, 0, unroll=False)


@jax.jit
def _run(kb_ids_seq, key_emb_table):
    table_lin = _relayout_table(key_emb_table.T, jnp.asarray(_E)).reshape(NPAD, DIM)
    idx_flat = kb_ids_seq.reshape(TOTAL)
    mesh = plsc.VectorSubcoreMesh(core_axis_name="c", subcore_axis_name="s")
    f = pl.kernel(
        _sc_body,
        out_type=jax.ShapeDtypeStruct((TOTAL, DIM), jnp.float32),
        mesh=mesh,
        scratch_types=[
            pltpu.VMEM((CHUNK,), jnp.int32),
            pltpu.VMEM((CHUNK,), jnp.int32),
            pltpu.VMEM((MAX_LEN, DIM), jnp.float32),
            pltpu.VMEM((CHUNK, DIM), jnp.float32),
            pltpu.SemaphoreType.DMA,
        ],
        compiler_params=pltpu.CompilerParams(use_tc_tiling_on_sc=False),
    )
    out = f(table_lin, idx_flat, jnp.asarray(_PE))
    return out.reshape(B, L, DIM)


def kernel(kb_ids_seq, key_emb_table):
    return _run(kb_ids_seq, key_emb_table)
